# baseline (speedup 1.0000x reference)
import jax
import jax.numpy as jnp
from jax import lax
from jax.experimental import pallas as pl
from jax.experimental.pallas import tpu as pltpu
from jax.experimental.pallas import tpu_sc as plsc

_N = 10000
_E = 320000
_D = 128
_DE = 16
_G = 256
_C = 128

NC = 2
NS = 16
NW = NC * NS

ECH = 128
NECH = _E // ECH
EITER = (NECH + NW - 1) // NW
EC2 = 80
NEC2 = _E // EC2
EITER2 = (NEC2 + NW - 1) // NW
PCH = 80
NPCH = _N // PCH
PITER = (NPCH + NW - 1) // NW
ST = 624
ST_TAIL_OFF = ST * NS
ST_TAIL = _N - ST_TAIL_OFF
GROWS_PT = _G // NS


def _striped(s, fn):
    fn(s * ST, ST)

    @pl.when(s == 0)
    def _():
        fn(ST_TAIL_OFF, ST_TAIL)

_BE = 512
_BN = 400


def _m0_body(ea, we, emb, w0, b0, o):
    r0 = jnp.dot(emb[...], w0[...], preferred_element_type=jnp.float32) + b0[...]
    e = jnp.dot(ea[...], we[...], preferred_element_type=jnp.float32)
    o[...] = jnp.maximum(e + r0, 0.0)


def _tc_m0(edge_attr, we0, emb, w0, b0):
    return pl.pallas_call(
        _m0_body,
        grid=(_E // _BE,),
        in_specs=[
            pl.BlockSpec((_BE, _DE), lambda i: (i, 0)),
            pl.BlockSpec((_DE, _D), lambda i: (0, 0)),
            pl.BlockSpec((1, _D), lambda i: (0, 0)),
            pl.BlockSpec((_D, _D), lambda i: (0, 0)),
            pl.BlockSpec((1, _D), lambda i: (0, 0)),
        ],
        out_specs=pl.BlockSpec((_BE, _D), lambda i: (i, 0)),
        out_shape=jax.ShapeDtypeStruct((_E, _D), jnp.float32),
    )(edge_attr, we0, emb, w0, b0)


def _e_body(ea, we, o):
    o[...] = jnp.dot(ea[...], we[...], preferred_element_type=jnp.float32)


def _tc_e(edge_attr, wel):
    return pl.pallas_call(
        _e_body,
        grid=(_E // _BE,),
        in_specs=[
            pl.BlockSpec((_BE, _DE), lambda i: (i, 0)),
            pl.BlockSpec((_DE, _D), lambda i: (0, 0)),
        ],
        out_specs=pl.BlockSpec((_BE, _D), lambda i: (i, 0)),
        out_shape=jax.ShapeDtypeStruct((_E, _D), jnp.float32),
    )(edge_attr, wel)


def _upd_body(aggp, degp, w, bb, o):
    a = aggp[0] + aggp[1]
    deg = degp[0, :, :1] + degp[1, :, :1]
    h = jnp.maximum(a / jnp.maximum(deg, 1.0), 0.0)
    o[...] = jnp.dot(h, w[...], preferred_element_type=jnp.float32) + bb[...]


def _tc_update(aggp, degp, wl, bl):
    return pl.pallas_call(
        _upd_body,
        grid=(_N // _BN,),
        in_specs=[
            pl.BlockSpec((NC, _BN, _D), lambda i: (0, i, 0)),
            pl.BlockSpec((NC, _BN, _D), lambda i: (0, i, 0)),
            pl.BlockSpec((_D, _D), lambda i: (0, 0)),
            pl.BlockSpec((1, _D), lambda i: (0, 0)),
        ],
        out_specs=pl.BlockSpec((_BN, _D), lambda i: (i, 0)),
        out_shape=jax.ShapeDtypeStruct((_N, _D), jnp.float32),
    )(aggp, degp, wl, bl)


def _hfin_body(aggp, degp, o):
    a = aggp[0] + aggp[1]
    deg = degp[0, :, :1] + degp[1, :, :1]
    o[...] = a / jnp.maximum(deg, 1.0)


def _tc_hfin(aggp, degp):
    return pl.pallas_call(
        _hfin_body,
        grid=(_N // _BN,),
        in_specs=[
            pl.BlockSpec((NC, _BN, _D), lambda i: (0, i, 0)),
            pl.BlockSpec((NC, _BN, _D), lambda i: (0, i, 0)),
        ],
        out_specs=pl.BlockSpec((_BN, _D), lambda i: (i, 0)),
        out_shape=jax.ShapeDtypeStruct((_N, _D), jnp.float32),
    )(aggp, degp)


def _final_body(g, c, pw, pbb, o):
    gm = (g[0] + g[1]) / jnp.maximum(c[0, :, :1] + c[1, :, :1], 1.0)
    o[...] = jnp.dot(gm, pw[...], preferred_element_type=jnp.float32) + pbb[...]


def _tc_final(gsum, cnt, pw, pbb):
    return pl.pallas_call(
        _final_body,
        grid=(1,),
        in_specs=[
            pl.BlockSpec((NC, _G, _D), lambda i: (0, 0, 0)),
            pl.BlockSpec((NC, _G, _D), lambda i: (0, 0, 0)),
            pl.BlockSpec((_D, _C), lambda i: (0, 0)),
            pl.BlockSpec((1, _C), lambda i: (0, 0)),
        ],
        out_specs=pl.BlockSpec((_G, _C), lambda i: (0, 0)),
        out_shape=jax.ShapeDtypeStruct((_G, _C), jnp.float32),
    )(gsum, cnt, pw, pbb)


def _sc_l0_body(m0_hbm, dst_hbm, zn_hbm, agg_out,
                idx0, idx1, buf0, buf1, semb0, semb1, sh_agg):
    c = lax.axis_index("c")
    s = lax.axis_index("s")
    w = s * NC + c
    idx = (idx0, idx1)
    buf = (buf0, buf1)
    semb = (semb0, semb1)
    _striped(s, lambda off, sz: pltpu.sync_copy(
        zn_hbm.at[pl.ds(off, sz)], sh_agg.at[pl.ds(off, sz)]))

    def prefetch(cid, p):
        base = cid * ECH
        pltpu.sync_copy(dst_hbm.at[pl.ds(base, ECH)], idx[p])
        pltpu.async_copy(m0_hbm.at[pl.ds(base, ECH)], buf[p], semb[p])

    prefetch(w, 0)
    plsc.subcore_barrier()

    def pair(kk, carry):
        for r in range(2):
            cid = w + NW * (kk * 2 + r)

            @pl.when(cid + NW < NECH)
            def _():
                prefetch(cid + NW, 1 - r)

            @pl.when(cid < NECH)
            def _():
                base = cid * ECH
                pltpu.make_async_copy(
                    m0_hbm.at[pl.ds(base, ECH)], buf[r], semb[r]).wait()
                pltpu.sync_copy(buf[r], sh_agg.at[idx[r]], add=True)

        return carry

    lax.fori_loop(0, (EITER + 1) // 2, pair, 0)
    plsc.subcore_barrier()
    _striped(s, lambda off, sz: pltpu.sync_copy(
        sh_agg.at[pl.ds(off, sz)], agg_out.at[c, pl.ds(off, sz)]))


def _sc_l0(m0, dst, zn):
    mesh = plsc.VectorSubcoreMesh(core_axis_name="c", subcore_axis_name="s", num_cores=NC, num_subcores=NS)
    f = pl.kernel(
        _sc_l0_body,
        out_type=jax.ShapeDtypeStruct((NC, _N, _D), jnp.float32),
        mesh=mesh,
        scratch_types=[
            pltpu.VMEM((ECH,), jnp.int32),
            pltpu.VMEM((ECH,), jnp.int32),
            pltpu.VMEM((ECH, _D), jnp.float32),
            pltpu.VMEM((ECH, _D), jnp.float32),
            pltpu.SemaphoreType.DMA,
            pltpu.SemaphoreType.DMA,
            pltpu.VMEM_SHARED((_N, _D), jnp.float32),
        ],
    )
    return f(m0, dst, zn)


def _sc_deg_body(dst_hbm, zn_hbm, ones_hbm, deg_out,
                 idx0, idx1, ones_v, semc0, semc1, sh_deg):
    c = lax.axis_index("c")
    s = lax.axis_index("s")
    w = s * NC + c
    idx = (idx0, idx1)
    _striped(s, lambda off, sz: pltpu.sync_copy(
        zn_hbm.at[pl.ds(off, sz)], sh_deg.at[pl.ds(off, sz)]))
    pltpu.sync_copy(ones_hbm, ones_v)
    pltpu.sync_copy(dst_hbm.at[pl.ds(w * ECH, ECH)], idx0)
    plsc.subcore_barrier()

    def pair(kk, carry):
        for r in range(2):
            cid = w + NW * (kk * 2 + r)

            @pl.when(cid + NW < NECH)
            def _():
                pltpu.sync_copy(
                    dst_hbm.at[pl.ds((cid + NW) * ECH, ECH)], idx[1 - r])

            @pl.when(cid < NECH)
            def _():
                pltpu.sync_copy(ones_v, sh_deg.at[idx[r]], add=True)

        return carry

    lax.fori_loop(0, (EITER + 1) // 2, pair, 0)
    plsc.subcore_barrier()
    _striped(s, lambda off, sz: pltpu.sync_copy(
        sh_deg.at[pl.ds(off, sz)], deg_out.at[c, pl.ds(off, sz)]))


def _sc_deg(dst, zn, ones):
    mesh = plsc.VectorSubcoreMesh(core_axis_name="c", subcore_axis_name="s", num_cores=NC, num_subcores=NS)
    f = pl.kernel(
        _sc_deg_body,
        out_type=jax.ShapeDtypeStruct((NC, _N, _D), jnp.float32),
        mesh=mesh,
        scratch_types=[
            pltpu.VMEM((ECH,), jnp.int32),
            pltpu.VMEM((ECH,), jnp.int32),
            pltpu.VMEM((ECH, _D), jnp.float32),
            pltpu.SemaphoreType.DMA,
            pltpu.SemaphoreType.DMA,
            pltpu.VMEM_SHARED((_N, _D), jnp.float32),
        ],
    )
    return f(dst, zn, ones)


def _sc_edge_body(hl_hbm, e_hbm, src_hbm, dst_hbm, zn_hbm, agg_out,
                  sidx0, sidx1, didx0, didx1, gbuf0, gbuf1, ebuf0, ebuf1,
                  semg0, semg1, seme0, seme1, sh_agg):
    c = lax.axis_index("c")
    s = lax.axis_index("s")
    w = s * NC + c
    sidx = (sidx0, sidx1)
    didx = (didx0, didx1)
    gbuf = (gbuf0, gbuf1)
    ebuf = (ebuf0, ebuf1)
    semg = (semg0, semg1)
    seme = (seme0, seme1)
    _striped(s, lambda off, sz: pltpu.sync_copy(
        zn_hbm.at[pl.ds(off, sz)], sh_agg.at[pl.ds(off, sz)]))

    def prefetch(cid, p):
        base = cid * EC2
        pltpu.sync_copy(src_hbm.at[pl.ds(base, EC2)], sidx[p])
        pltpu.sync_copy(dst_hbm.at[pl.ds(base, EC2)], didx[p])
        pltpu.async_copy(hl_hbm.at[sidx[p]], gbuf[p], semg[p])
        pltpu.async_copy(e_hbm.at[pl.ds(base, EC2)], ebuf[p], seme[p])

    prefetch(w, 0)
    plsc.subcore_barrier()

    def pair(kk, carry):
        for r in range(2):
            cid = w + NW * (kk * 2 + r)

            @pl.when(cid + NW < NEC2)
            def _():
                prefetch(cid + NW, 1 - r)

            @pl.when(cid < NEC2)
            def _():
                base = cid * EC2
                pltpu.make_async_copy(hl_hbm.at[sidx[r]], gbuf[r], semg[r]).wait()
                pltpu.make_async_copy(
                    e_hbm.at[pl.ds(base, EC2)], ebuf[r], seme[r]).wait()

                def row(rr, cc):
                    for j in range(_D // 16):
                        sl = pl.ds(j * 16, 16)
                        ebuf[r][rr, sl] = jnp.maximum(
                            ebuf[r][rr, sl] + gbuf[r][rr, sl], 0.0)
                    return cc

                lax.fori_loop(0, EC2, row, 0)
                pltpu.sync_copy(ebuf[r], sh_agg.at[didx[r]], add=True)

        return carry

    lax.fori_loop(0, (EITER2 + 1) // 2, pair, 0)
    plsc.subcore_barrier()
    _striped(s, lambda off, sz: pltpu.sync_copy(
        sh_agg.at[pl.ds(off, sz)], agg_out.at[c, pl.ds(off, sz)]))


def _sc_edge(hl, e, src, dst, zn):
    mesh = plsc.VectorSubcoreMesh(core_axis_name="c", subcore_axis_name="s", num_cores=NC, num_subcores=NS)
    f = pl.kernel(
        _sc_edge_body,
        out_type=jax.ShapeDtypeStruct((NC, _N, _D), jnp.float32),
        mesh=mesh,
        scratch_types=[
            pltpu.VMEM((EC2,), jnp.int32),
            pltpu.VMEM((EC2,), jnp.int32),
            pltpu.VMEM((EC2,), jnp.int32),
            pltpu.VMEM((EC2,), jnp.int32),
            pltpu.VMEM((EC2, _D), jnp.float32),
            pltpu.VMEM((EC2, _D), jnp.float32),
            pltpu.VMEM((EC2, _D), jnp.float32),
            pltpu.VMEM((EC2, _D), jnp.float32),
            pltpu.SemaphoreType.DMA,
            pltpu.SemaphoreType.DMA,
            pltpu.SemaphoreType.DMA,
            pltpu.SemaphoreType.DMA,
            pltpu.VMEM_SHARED((_N, _D), jnp.float32),
        ],
    )
    return f(hl, e, src, dst, zn)


def _sc_pool_body(h_hbm, batch_hbm, zg_hbm, ones_hbm, gsum_out, cnt_out,
                  bidx, hbuf, ones_v, sh_g, sh_c):
    c = lax.axis_index("c")
    s = lax.axis_index("s")
    w = s * NC + c
    pltpu.sync_copy(zg_hbm.at[pl.ds(s * GROWS_PT, GROWS_PT)],
                    sh_g.at[pl.ds(s * GROWS_PT, GROWS_PT)])
    pltpu.sync_copy(zg_hbm.at[pl.ds(s * GROWS_PT, GROWS_PT)],
                    sh_c.at[pl.ds(s * GROWS_PT, GROWS_PT)])
    pltpu.sync_copy(ones_hbm, ones_v)
    plsc.subcore_barrier()

    def chunk(i, carry):
        cid = w + NW * i

        @pl.when(cid < NPCH)
        def _():
            base = cid * PCH
            pltpu.sync_copy(batch_hbm.at[pl.ds(base, PCH)], bidx)
            pltpu.sync_copy(h_hbm.at[pl.ds(base, PCH)], hbuf)
            pltpu.sync_copy(hbuf, sh_g.at[bidx], add=True)
            pltpu.sync_copy(ones_v, sh_c.at[bidx], add=True)

        return carry

    lax.fori_loop(0, PITER, chunk, 0)
    plsc.subcore_barrier()
    pltpu.sync_copy(sh_g.at[pl.ds(s * GROWS_PT, GROWS_PT)],
                    gsum_out.at[c, pl.ds(s * GROWS_PT, GROWS_PT)])
    pltpu.sync_copy(sh_c.at[pl.ds(s * GROWS_PT, GROWS_PT)],
                    cnt_out.at[c, pl.ds(s * GROWS_PT, GROWS_PT)])


def _sc_pool(h, batch, zg, ones):
    mesh = plsc.VectorSubcoreMesh(core_axis_name="c", subcore_axis_name="s", num_cores=NC, num_subcores=NS)
    f = pl.kernel(
        _sc_pool_body,
        out_type=(
            jax.ShapeDtypeStruct((NC, _G, _D), jnp.float32),
            jax.ShapeDtypeStruct((NC, _G, _D), jnp.float32),
        ),
        mesh=mesh,
        scratch_types=[
            pltpu.VMEM((PCH,), jnp.int32),
            pltpu.VMEM((PCH, _D), jnp.float32),
            pltpu.VMEM((PCH, _D), jnp.float32),
            pltpu.VMEM_SHARED((_G, _D), jnp.float32),
            pltpu.VMEM_SHARED((_G, _D), jnp.float32),
        ],
    )
    return f(h, batch, zg, ones)


def kernel(x, edge_index, edge_attr, batch, node_emb, W, We, b, pW, pb):
    src = edge_index[0]
    dst = edge_index[1]
    zn = jnp.zeros((_N, _D), jnp.float32)
    onese = jnp.ones((ECH, _D), jnp.float32)
    onesp = jnp.ones((PCH, _D), jnp.float32)

    degp = _sc_deg(dst, zn, onese)
    m0 = _tc_m0(edge_attr, We[0], node_emb, W[0], b[0].reshape(1, _D))
    e1 = _tc_e(edge_attr, We[1])
    e2 = _tc_e(edge_attr, We[2])
    aggp = _sc_l0(m0, dst, zn)

    hl = _tc_update(aggp, degp, W[1], b[1].reshape(1, _D))
    aggp = _sc_edge(hl, e1, src, dst, zn)

    hl = _tc_update(aggp, degp, W[2], b[2].reshape(1, _D))
    aggp = _sc_edge(hl, e2, src, dst, zn)

    hfin = _tc_hfin(aggp, degp)
    gsum, cnt = _sc_pool(hfin, batch, zn[:_G], onesp)
    out = _tc_final(gsum, cnt, pW, pb.reshape(1, _C))
    return out


# speedup vs baseline: 1.4767x; 1.4767x over previous
import jax
import jax.numpy as jnp
from jax import lax
from jax.experimental import pallas as pl
from jax.experimental.pallas import tpu as pltpu
from jax.experimental.pallas import tpu_sc as plsc

_N = 10000
_E = 320000
_D = 128
_DE = 16
_G = 256
_C = 128

NC = 2
NS = 16
NW = NC * NS

ECH = 128
NECH = _E // ECH
EITER = (NECH + NW - 1) // NW
EC2 = 80
NEC2 = _E // EC2
EITER2 = (NEC2 + NW - 1) // NW
PCH = 80
NPCH = _N // PCH
PITER = (NPCH + NW - 1) // NW
ST = 624
ST_TAIL_OFF = ST * NS
ST_TAIL = _N - ST_TAIL_OFF
GROWS_PT = _G // NS


def _striped(s, fn):
    fn(s * ST, ST)

    @pl.when(s == 0)
    def _():
        fn(ST_TAIL_OFF, ST_TAIL)

_BE = 3200
_BN = 400


def _m0_body(ea, we, emb, w0, b0, o):
    r0 = jnp.dot(emb[...], w0[...], preferred_element_type=jnp.float32) + b0[...]
    e = jnp.dot(ea[...], we[...], preferred_element_type=jnp.float32)
    o[...] = jnp.maximum(e + r0, 0.0)


def _tc_m0(edge_attr, we0, emb, w0, b0):
    return pl.pallas_call(
        _m0_body,
        grid=(_E // _BE,),
        in_specs=[
            pl.BlockSpec((_BE, _DE), lambda i: (i, 0)),
            pl.BlockSpec((_DE, _D), lambda i: (0, 0)),
            pl.BlockSpec((1, _D), lambda i: (0, 0)),
            pl.BlockSpec((_D, _D), lambda i: (0, 0)),
            pl.BlockSpec((1, _D), lambda i: (0, 0)),
        ],
        out_specs=pl.BlockSpec((_BE, _D), lambda i: (i, 0)),
        out_shape=jax.ShapeDtypeStruct((_E, _D), jnp.float32),
    )(edge_attr, we0, emb, w0, b0)


def _e_body(ea, we, o):
    o[...] = jnp.dot(ea[...], we[...], preferred_element_type=jnp.float32)


def _tc_e(edge_attr, wel):
    return pl.pallas_call(
        _e_body,
        grid=(_E // _BE,),
        in_specs=[
            pl.BlockSpec((_BE, _DE), lambda i: (i, 0)),
            pl.BlockSpec((_DE, _D), lambda i: (0, 0)),
        ],
        out_specs=pl.BlockSpec((_BE, _D), lambda i: (i, 0)),
        out_shape=jax.ShapeDtypeStruct((_E, _D), jnp.float32),
    )(edge_attr, wel)


def _upd_body(aggp, degp, w, bb, o):
    a = aggp[0] + aggp[1]
    deg = degp[0, :, :1] + degp[1, :, :1]
    h = jnp.maximum(a / jnp.maximum(deg, 1.0), 0.0)
    o[...] = jnp.dot(h, w[...], preferred_element_type=jnp.float32) + bb[...]


def _tc_update(aggp, degp, wl, bl):
    return pl.pallas_call(
        _upd_body,
        grid=(_N // _BN,),
        in_specs=[
            pl.BlockSpec((NC, _BN, _D), lambda i: (0, i, 0)),
            pl.BlockSpec((NC, _BN, _D), lambda i: (0, i, 0)),
            pl.BlockSpec((_D, _D), lambda i: (0, 0)),
            pl.BlockSpec((1, _D), lambda i: (0, 0)),
        ],
        out_specs=pl.BlockSpec((_BN, _D), lambda i: (i, 0)),
        out_shape=jax.ShapeDtypeStruct((_N, _D), jnp.float32),
    )(aggp, degp, wl, bl)


def _hfin_body(aggp, degp, o):
    a = aggp[0] + aggp[1]
    deg = degp[0, :, :1] + degp[1, :, :1]
    o[...] = a / jnp.maximum(deg, 1.0)


def _tc_hfin(aggp, degp):
    return pl.pallas_call(
        _hfin_body,
        grid=(_N // _BN,),
        in_specs=[
            pl.BlockSpec((NC, _BN, _D), lambda i: (0, i, 0)),
            pl.BlockSpec((NC, _BN, _D), lambda i: (0, i, 0)),
        ],
        out_specs=pl.BlockSpec((_BN, _D), lambda i: (i, 0)),
        out_shape=jax.ShapeDtypeStruct((_N, _D), jnp.float32),
    )(aggp, degp)


def _final_body(g, c, pw, pbb, o):
    gm = (g[0] + g[1]) / jnp.maximum(c[0, :, :1] + c[1, :, :1], 1.0)
    o[...] = jnp.dot(gm, pw[...], preferred_element_type=jnp.float32) + pbb[...]


def _tc_final(gsum, cnt, pw, pbb):
    return pl.pallas_call(
        _final_body,
        grid=(1,),
        in_specs=[
            pl.BlockSpec((NC, _G, _D), lambda i: (0, 0, 0)),
            pl.BlockSpec((NC, _G, _D), lambda i: (0, 0, 0)),
            pl.BlockSpec((_D, _C), lambda i: (0, 0)),
            pl.BlockSpec((1, _C), lambda i: (0, 0)),
        ],
        out_specs=pl.BlockSpec((_G, _C), lambda i: (0, 0)),
        out_shape=jax.ShapeDtypeStruct((_G, _C), jnp.float32),
    )(gsum, cnt, pw, pbb)


def _sc_l0_body(m0_hbm, dst_hbm, zn_hbm, agg_out,
                idx0, idx1, buf0, buf1, semb0, semb1, sh_agg):
    c = lax.axis_index("c")
    s = lax.axis_index("s")
    w = s * NC + c
    idx = (idx0, idx1)
    buf = (buf0, buf1)
    semb = (semb0, semb1)
    _striped(s, lambda off, sz: pltpu.sync_copy(
        zn_hbm.at[pl.ds(off, sz)], sh_agg.at[pl.ds(off, sz)]))

    def prefetch(cid, p):
        base = cid * ECH
        pltpu.sync_copy(dst_hbm.at[pl.ds(base, ECH)], idx[p])
        pltpu.async_copy(m0_hbm.at[pl.ds(base, ECH)], buf[p], semb[p])

    prefetch(w, 0)
    plsc.subcore_barrier()

    def pair(kk, carry):
        for r in range(2):
            cid = w + NW * (kk * 2 + r)

            @pl.when(cid + NW < NECH)
            def _():
                prefetch(cid + NW, 1 - r)

            @pl.when(cid < NECH)
            def _():
                base = cid * ECH
                pltpu.make_async_copy(
                    m0_hbm.at[pl.ds(base, ECH)], buf[r], semb[r]).wait()
                pltpu.sync_copy(buf[r], sh_agg.at[idx[r]], add=True)

        return carry

    lax.fori_loop(0, (EITER + 1) // 2, pair, 0)
    plsc.subcore_barrier()
    _striped(s, lambda off, sz: pltpu.sync_copy(
        sh_agg.at[pl.ds(off, sz)], agg_out.at[c, pl.ds(off, sz)]))


def _sc_l0(m0, dst, zn):
    mesh = plsc.VectorSubcoreMesh(core_axis_name="c", subcore_axis_name="s", num_cores=NC, num_subcores=NS)
    f = pl.kernel(
        _sc_l0_body,
        out_type=jax.ShapeDtypeStruct((NC, _N, _D), jnp.float32),
        mesh=mesh,
        scratch_types=[
            pltpu.VMEM((ECH,), jnp.int32),
            pltpu.VMEM((ECH,), jnp.int32),
            pltpu.VMEM((ECH, _D), jnp.float32),
            pltpu.VMEM((ECH, _D), jnp.float32),
            pltpu.SemaphoreType.DMA,
            pltpu.SemaphoreType.DMA,
            pltpu.VMEM_SHARED((_N, _D), jnp.float32),
        ],
    )
    return f(m0, dst, zn)


def _sc_deg_body(dst_hbm, zn_hbm, ones_hbm, deg_out,
                 idx0, idx1, ones_v, semc0, semc1, sh_deg):
    c = lax.axis_index("c")
    s = lax.axis_index("s")
    w = s * NC + c
    idx = (idx0, idx1)
    _striped(s, lambda off, sz: pltpu.sync_copy(
        zn_hbm.at[pl.ds(off, sz)], sh_deg.at[pl.ds(off, sz)]))
    pltpu.sync_copy(ones_hbm, ones_v)
    pltpu.sync_copy(dst_hbm.at[pl.ds(w * ECH, ECH)], idx0)
    plsc.subcore_barrier()

    def pair(kk, carry):
        for r in range(2):
            cid = w + NW * (kk * 2 + r)

            @pl.when(cid + NW < NECH)
            def _():
                pltpu.sync_copy(
                    dst_hbm.at[pl.ds((cid + NW) * ECH, ECH)], idx[1 - r])

            @pl.when(cid < NECH)
            def _():
                pltpu.sync_copy(ones_v, sh_deg.at[idx[r]], add=True)

        return carry

    lax.fori_loop(0, (EITER + 1) // 2, pair, 0)
    plsc.subcore_barrier()
    _striped(s, lambda off, sz: pltpu.sync_copy(
        sh_deg.at[pl.ds(off, sz)], deg_out.at[c, pl.ds(off, sz)]))


def _sc_deg(dst, zn, ones):
    mesh = plsc.VectorSubcoreMesh(core_axis_name="c", subcore_axis_name="s", num_cores=NC, num_subcores=NS)
    f = pl.kernel(
        _sc_deg_body,
        out_type=jax.ShapeDtypeStruct((NC, _N, _D), jnp.float32),
        mesh=mesh,
        scratch_types=[
            pltpu.VMEM((ECH,), jnp.int32),
            pltpu.VMEM((ECH,), jnp.int32),
            pltpu.VMEM((ECH, _D), jnp.float32),
            pltpu.SemaphoreType.DMA,
            pltpu.SemaphoreType.DMA,
            pltpu.VMEM_SHARED((_N, _D), jnp.float32),
        ],
    )
    return f(dst, zn, ones)


def _sc_edge_body(hl_hbm, e_hbm, src_hbm, dst_hbm, zn_hbm, agg_out,
                  sidx0, sidx1, didx0, didx1, gbuf0, gbuf1, ebuf0, ebuf1,
                  semg0, semg1, seme0, seme1, sh_agg):
    c = lax.axis_index("c")
    s = lax.axis_index("s")
    w = s * NC + c
    sidx = (sidx0, sidx1)
    didx = (didx0, didx1)
    gbuf = (gbuf0, gbuf1)
    ebuf = (ebuf0, ebuf1)
    semg = (semg0, semg1)
    seme = (seme0, seme1)
    _striped(s, lambda off, sz: pltpu.sync_copy(
        zn_hbm.at[pl.ds(off, sz)], sh_agg.at[pl.ds(off, sz)]))

    def prefetch(cid, p):
        base = cid * EC2
        pltpu.sync_copy(src_hbm.at[pl.ds(base, EC2)], sidx[p])
        pltpu.sync_copy(dst_hbm.at[pl.ds(base, EC2)], didx[p])
        pltpu.async_copy(hl_hbm.at[sidx[p]], gbuf[p], semg[p])
        pltpu.async_copy(e_hbm.at[pl.ds(base, EC2)], ebuf[p], seme[p])

    prefetch(w, 0)
    plsc.subcore_barrier()

    def pair(kk, carry):
        for r in range(2):
            cid = w + NW * (kk * 2 + r)

            @pl.when(cid + NW < NEC2)
            def _():
                prefetch(cid + NW, 1 - r)

            @pl.when(cid < NEC2)
            def _():
                base = cid * EC2
                pltpu.make_async_copy(hl_hbm.at[sidx[r]], gbuf[r], semg[r]).wait()
                pltpu.make_async_copy(
                    e_hbm.at[pl.ds(base, EC2)], ebuf[r], seme[r]).wait()

                def row(rr, cc):
                    for j in range(_D // 16):
                        sl = pl.ds(j * 16, 16)
                        ebuf[r][rr, sl] = jnp.maximum(
                            ebuf[r][rr, sl] + gbuf[r][rr, sl], 0.0)
                    return cc

                lax.fori_loop(0, EC2, row, 0)
                pltpu.sync_copy(ebuf[r], sh_agg.at[didx[r]], add=True)

        return carry

    lax.fori_loop(0, (EITER2 + 1) // 2, pair, 0)
    plsc.subcore_barrier()
    _striped(s, lambda off, sz: pltpu.sync_copy(
        sh_agg.at[pl.ds(off, sz)], agg_out.at[c, pl.ds(off, sz)]))


def _sc_edge(hl, e, src, dst, zn):
    mesh = plsc.VectorSubcoreMesh(core_axis_name="c", subcore_axis_name="s", num_cores=NC, num_subcores=NS)
    f = pl.kernel(
        _sc_edge_body,
        out_type=jax.ShapeDtypeStruct((NC, _N, _D), jnp.float32),
        mesh=mesh,
        scratch_types=[
            pltpu.VMEM((EC2,), jnp.int32),
            pltpu.VMEM((EC2,), jnp.int32),
            pltpu.VMEM((EC2,), jnp.int32),
            pltpu.VMEM((EC2,), jnp.int32),
            pltpu.VMEM((EC2, _D), jnp.float32),
            pltpu.VMEM((EC2, _D), jnp.float32),
            pltpu.VMEM((EC2, _D), jnp.float32),
            pltpu.VMEM((EC2, _D), jnp.float32),
            pltpu.SemaphoreType.DMA,
            pltpu.SemaphoreType.DMA,
            pltpu.SemaphoreType.DMA,
            pltpu.SemaphoreType.DMA,
            pltpu.VMEM_SHARED((_N, _D), jnp.float32),
        ],
    )
    return f(hl, e, src, dst, zn)


def _sc_pool_body(h_hbm, batch_hbm, zg_hbm, ones_hbm, gsum_out, cnt_out,
                  bidx, hbuf, ones_v, sh_g, sh_c):
    c = lax.axis_index("c")
    s = lax.axis_index("s")
    w = s * NC + c
    pltpu.sync_copy(zg_hbm.at[pl.ds(s * GROWS_PT, GROWS_PT)],
                    sh_g.at[pl.ds(s * GROWS_PT, GROWS_PT)])
    pltpu.sync_copy(zg_hbm.at[pl.ds(s * GROWS_PT, GROWS_PT)],
                    sh_c.at[pl.ds(s * GROWS_PT, GROWS_PT)])
    pltpu.sync_copy(ones_hbm, ones_v)
    plsc.subcore_barrier()

    def chunk(i, carry):
        cid = w + NW * i

        @pl.when(cid < NPCH)
        def _():
            base = cid * PCH
            pltpu.sync_copy(batch_hbm.at[pl.ds(base, PCH)], bidx)
            pltpu.sync_copy(h_hbm.at[pl.ds(base, PCH)], hbuf)
            pltpu.sync_copy(hbuf, sh_g.at[bidx], add=True)
            pltpu.sync_copy(ones_v, sh_c.at[bidx], add=True)

        return carry

    lax.fori_loop(0, PITER, chunk, 0)
    plsc.subcore_barrier()
    pltpu.sync_copy(sh_g.at[pl.ds(s * GROWS_PT, GROWS_PT)],
                    gsum_out.at[c, pl.ds(s * GROWS_PT, GROWS_PT)])
    pltpu.sync_copy(sh_c.at[pl.ds(s * GROWS_PT, GROWS_PT)],
                    cnt_out.at[c, pl.ds(s * GROWS_PT, GROWS_PT)])


def _sc_pool(h, batch, zg, ones):
    mesh = plsc.VectorSubcoreMesh(core_axis_name="c", subcore_axis_name="s", num_cores=NC, num_subcores=NS)
    f = pl.kernel(
        _sc_pool_body,
        out_type=(
            jax.ShapeDtypeStruct((NC, _G, _D), jnp.float32),
            jax.ShapeDtypeStruct((NC, _G, _D), jnp.float32),
        ),
        mesh=mesh,
        scratch_types=[
            pltpu.VMEM((PCH,), jnp.int32),
            pltpu.VMEM((PCH, _D), jnp.float32),
            pltpu.VMEM((PCH, _D), jnp.float32),
            pltpu.VMEM_SHARED((_G, _D), jnp.float32),
            pltpu.VMEM_SHARED((_G, _D), jnp.float32),
        ],
    )
    return f(h, batch, zg, ones)


def kernel(x, edge_index, edge_attr, batch, node_emb, W, We, b, pW, pb):
    src = edge_index[0]
    dst = edge_index[1]
    zn = jnp.zeros((_N, _D), jnp.float32)
    onese = jnp.ones((ECH, _D), jnp.float32)
    onesp = jnp.ones((PCH, _D), jnp.float32)

    degp = _sc_deg(dst, zn, onese)
    m0 = _tc_m0(edge_attr, We[0], node_emb, W[0], b[0].reshape(1, _D))
    e1 = _tc_e(edge_attr, We[1])
    e2 = _tc_e(edge_attr, We[2])
    aggp = _sc_l0(m0, dst, zn)

    hl = _tc_update(aggp, degp, W[1], b[1].reshape(1, _D))
    aggp = _sc_edge(hl, e1, src, dst, zn)

    hl = _tc_update(aggp, degp, W[2], b[2].reshape(1, _D))
    aggp = _sc_edge(hl, e2, src, dst, zn)

    hfin = _tc_hfin(aggp, degp)
    gsum, cnt = _sc_pool(hfin, batch, zn[:_G], onesp)
    out = _tc_final(gsum, cnt, pW, pb.reshape(1, _C))
    return out
